# baseline (speedup 1.0000x reference)
import functools

import jax
import jax.numpy as jnp
from jax import lax
from jax.experimental import pallas as pl
from jax.experimental.pallas import tpu as pltpu
from jax.experimental.pallas import tpu_sc as plsc

N0, N1, N2 = 10000, 320000, 160000
C, EMB, NREG = 128, 32, 64

LANES = 16
BATCH = 128
CBUF = 160
CHUNK = 2000


def _sc_spmm_call(mats, n_out, rr, rounds):
    n_pad = 2 * rounds * rr
    num_mats = len(mats)
    npts = []
    for rows, cols, vals, table in mats:
        nnz = rows.shape[0]
        assert nnz % (16 * CHUNK) == 0, nnz
        npts.append(nnz // 16)

    mesh = plsc.VectorSubcoreMesh(core_axis_name="c", subcore_axis_name="s",
                                  num_cores=2, num_subcores=16)
    rpt = rr // 16
    nwb = rpt // BATCH

    def body(*refs):
        mat_refs = refs[:4 * num_mats]
        out_hbm = refs[4 * num_mats]
        (rbuf, cbuf, vbuf, kr, kc, kv, ridx, cidx, grows,
         acc, sem) = refs[4 * num_mats + 1:]
        c = lax.axis_index("c")
        s = lax.axis_index("s")

        def make_batch(table_ref):
            def do_batch(b, m):
                base = b * BATCH
                limit = m - base
                for q in range(BATCH // LANES):
                    d = pl.ds(q * LANES, LANES)
                    dk = pl.ds(base + q * LANES, LANES)
                    lane = q * LANES + lax.iota(jnp.int32, LANES)
                    valid = lane < limit
                    cidx[d] = jnp.where(valid, kc[dk], 0)
                    ridx[d] = jnp.where(valid, kr[dk], rr)
                pltpu.async_copy(table_ref.at[cidx], grows, sem).wait()

                def scale(j16, _):
                    vv = kv[pl.ds(base + j16 * LANES, LANES)]
                    for l in range(LANES):
                        j = j16 * LANES + l
                        v = vv[l]
                        for q in range(C // LANES):
                            d = pl.ds(q * LANES, LANES)
                            grows[j, d] = grows[j, d] * v
                    return 0
                lax.fori_loop(0, BATCH // LANES, scale, 0)
                pltpu.sync_copy(grows, acc.at[ridx], add=True)
                return 0
            return do_batch

        def round_body(rnd, _):
            lo = (2 * rnd + c) * rr
            def zfill(i, _):
                for q in range(C // LANES):
                    grows[i, pl.ds(q * LANES, LANES)] = jnp.zeros(
                        (LANES,), jnp.float32)
                return 0
            lax.fori_loop(0, BATCH, zfill, 0)
            for t in range(nwb):
                base = s * rpt + t * BATCH
                pltpu.sync_copy(grows, acc.at[pl.ds(base, BATCH)])
            plsc.subcore_barrier()

            for m in range(num_mats):
                rows_hbm, cols_hbm, vals_hbm, table_hbm = \
                    mat_refs[4 * m:4 * m + 4]
                npt = npts[m]
                do_batch = make_batch(table_hbm)

                def chunk_body(ch, _, rows_hbm=rows_hbm,
                               cols_hbm=cols_hbm, vals_hbm=vals_hbm,
                               npt=npt, do_batch=do_batch):
                    base = s * npt + ch * CHUNK
                    d = pl.ds(base, CHUNK)
                    pltpu.sync_copy(rows_hbm.at[d], rbuf)
                    pltpu.sync_copy(cols_hbm.at[d], cbuf)
                    pltpu.sync_copy(vals_hbm.at[d], vbuf)

                    def vec_body(g, off):
                        dg = pl.ds(g * LANES, LANES)
                        r = rbuf[dg]
                        mask = (r >= lo) & (r < lo + rr)
                        mi = jnp.where(mask, 1, 0).astype(jnp.int32)
                        incl = plsc.cumsum(mi)
                        dest = (off + incl) - mi
                        plsc.store_scatter(kr, [dest], r - lo, mask=mask)
                        plsc.store_scatter(kc, [dest], cbuf[dg], mask=mask)
                        plsc.store_scatter(kv, [dest], vbuf[dg], mask=mask)
                        return off + incl[LANES - 1]
                    m_cnt = lax.fori_loop(0, CHUNK // LANES, vec_body, 0)

                    nb = (m_cnt + (BATCH - 1)) // BATCH
                    lax.fori_loop(0, nb, lambda b, _: do_batch(b, m_cnt), 0)
                    return 0

                lax.fori_loop(0, npt // CHUNK, chunk_body, 0)
            plsc.subcore_barrier()

            pltpu.sync_copy(acc.at[pl.ds(s * rpt, rpt)],
                            out_hbm.at[pl.ds(lo + s * rpt, rpt)])
            return 0

        lax.fori_loop(0, rounds, round_body, 0)

    flat_in = []
    for rows, cols, vals, table in mats:
        flat_in += [rows, cols, vals, table]

    fn = pl.kernel(
        body,
        mesh=mesh,
        compiler_params=pltpu.CompilerParams(needs_layout_passes=False),
        out_type=jax.ShapeDtypeStruct((n_pad, C), jnp.float32),
        scratch_types=[
            pltpu.VMEM((CHUNK,), jnp.int32),
            pltpu.VMEM((CHUNK,), jnp.int32),
            pltpu.VMEM((CHUNK,), jnp.float32),
            pltpu.VMEM((CHUNK + 48,), jnp.int32),
            pltpu.VMEM((CHUNK + 48,), jnp.int32),
            pltpu.VMEM((CHUNK + 48,), jnp.float32),
            pltpu.VMEM((BATCH,), jnp.int32),
            pltpu.VMEM((BATCH,), jnp.int32),
            pltpu.VMEM((BATCH, C), jnp.float32),
            pltpu.VMEM_SHARED((rr + 8, C), jnp.float32),
            pltpu.SemaphoreType.DMA,
        ],
    )
    out = fn(*flat_in)
    return out[:n_out]


def _mm_body(nout, relu_in, *refs):
    x_ref = refs[0]
    w_refs = refs[1:1 + nout]
    o_refs = refs[1 + nout:]
    x = x_ref[...]
    if relu_in:
        x = jnp.maximum(x, 0.0)
    for w_ref, o_ref in zip(w_refs, o_refs):
        o_ref[...] = jnp.dot(x, w_ref[...],
                             preferred_element_type=jnp.float32)


def _mm_multi(x, ws, relu_in=False, block_rows=2048):
    n = x.shape[0]
    n_pad = ((n + block_rows - 1) // block_rows) * block_rows
    if n_pad != n:
        x = jnp.pad(x, ((0, n_pad - n), (0, 0)))
    k = len(ws)
    outs = pl.pallas_call(
        functools.partial(_mm_body, k, relu_in),
        grid=(n_pad // block_rows,),
        in_specs=([pl.BlockSpec((block_rows, C), lambda i: (i, 0))]
                  + [pl.BlockSpec((C, C), lambda i: (0, 0))
                     for _ in range(k)]),
        out_specs=[pl.BlockSpec((block_rows, C), lambda i: (i, 0))
                   for _ in range(k)],
        out_shape=[jax.ShapeDtypeStruct((n_pad, C), jnp.float32)
                   for _ in range(k)],
    )(x, *ws)
    if n_pad != n:
        outs = [o[:n] for o in outs]
    return outs


def _head_body(x0_ref, m_ref, w_ref, b_ref, o_ref):
    x0 = jnp.maximum(x0_ref[...], 0.0)
    emb = jnp.dot(x0, w_ref[...],
                  preferred_element_type=jnp.float32) + b_ref[...]
    m = m_ref[...]
    counts = jnp.sum(m, axis=1, keepdims=True)
    pooled = jnp.dot(m, emb, preferred_element_type=jnp.float32)
    o_ref[...] = jnp.where(counts > 0, pooled / jnp.maximum(counts, 1.0), 0.0)


def _head(x0_pre, masks_f32, lin_w, lin_b):
    n = x0_pre.shape[0]
    return pl.pallas_call(
        _head_body,
        in_specs=[
            pl.BlockSpec((n, C), lambda: (0, 0)),
            pl.BlockSpec((NREG, n), lambda: (0, 0)),
            pl.BlockSpec((C, EMB), lambda: (0, 0)),
            pl.BlockSpec((1, EMB), lambda: (0, 0)),
        ],
        out_specs=pl.BlockSpec((NREG, EMB), lambda: (0, 0)),
        out_shape=jax.ShapeDtypeStruct((NREG, EMB), jnp.float32),
    )(x0_pre, masks_f32, lin_w, lin_b.reshape(1, EMB))


RR0, ROUNDS0 = 6144, 1
RR1, ROUNDS1 = 12288, 14


def kernel(x_0, x_1, x_2, incidence_1_indices, incidence_1_values,
           incidence_1_norm_indices, incidence_1_norm_values,
           incidence_2_indices, incidence_2_values,
           incidence_2_norm_indices, incidence_2_norm_values,
           adjacency_up_0_norm_indices, adjacency_up_0_norm_values,
           adjacency_up_1_norm_indices, adjacency_up_1_norm_values,
           adjacency_down_1_norm_indices, adjacency_down_1_norm_values,
           adjacency_down_2_norm_indices, adjacency_down_2_norm_values,
           masks_ip, params):
    p0 = {k[3:]: v for k, v in params.items() if k.startswith('l0_')}
    p1 = {k[3:]: v for k, v in params.items() if k.startswith('l1_')}

    def rcv(idx, vals):
        return jnp.asarray(idx[0]), jnp.asarray(idx[1]), vals

    a00 = rcv(adjacency_up_0_norm_indices, adjacency_up_0_norm_values)
    b1 = rcv(incidence_1_indices, incidence_1_values)
    b1n = rcv(incidence_1_norm_indices, incidence_1_norm_values)
    a1d = rcv(adjacency_down_1_norm_indices, adjacency_down_1_norm_values)
    a1u = rcv(adjacency_up_1_norm_indices, adjacency_up_1_norm_values)
    b2 = rcv(incidence_2_indices, incidence_2_values)

    t00, t01 = _mm_multi(x_0, [p0['W00'], p0['W01']])
    t10, t11d, t11u = _mm_multi(x_1, [p0['W10'], p0['W11d'], p0['W11u']])
    t21, = _mm_multi(x_2, [p0['W21']])

    o0 = _sc_spmm_call([a00 + (t00,), b1 + (t10,)], N0, RR0, ROUNDS0)
    o1 = _sc_spmm_call([b1n + (t01,), a1d + (t11d,), a1u + (t11u,),
                        b2 + (t21,)], N1, RR1, ROUNDS1)

    s00, = _mm_multi(o0, [p1['W00']], relu_in=True)
    s10, = _mm_multi(o1, [p1['W10']], relu_in=True)
    o0b = _sc_spmm_call([a00 + (s00,), b1 + (s10,)], N0, RR0, ROUNDS0)

    return _head(o0b, masks_ip.astype(jnp.float32), params['lin0_w'],
                 params['lin0_b'])


# speedup vs baseline: 3.3653x; 3.3653x over previous
import functools

import jax
import jax.numpy as jnp
from jax import lax
from jax.experimental import pallas as pl
from jax.experimental.pallas import tpu as pltpu
from jax.experimental.pallas import tpu_sc as plsc

N0, N1, N2 = 10000, 320000, 160000
C, EMB, NREG = 128, 32, 64

LANES = 16
BATCH = 128
CBUF = 160
CHUNK = 2000


def _sc_spmm_call(mats, n_out, rr, rounds):
    n_pad = 2 * rounds * rr
    num_mats = len(mats)
    npts = []
    for rows, cols, vals, table in mats:
        nnz = rows.shape[0]
        assert nnz % (16 * CHUNK) == 0, nnz
        npts.append(nnz // 16)

    mesh = plsc.VectorSubcoreMesh(core_axis_name="c", subcore_axis_name="s",
                                  num_cores=2, num_subcores=16)
    rpt = rr // 16
    nwb = rpt // BATCH

    def body(*refs):
        mat_refs = refs[:2 * num_mats]
        out_hbm = refs[2 * num_mats]
        (sbuf, crb, ccb, cvb, ridx, cidx, grows,
         acc, sem, sem2) = refs[2 * num_mats + 1:]
        c = lax.axis_index("c")
        s = lax.axis_index("s")
        iota3 = lax.iota(jnp.int32, LANES) * 3

        def drain_scatter():
            pltpu.make_async_copy(grows, acc.at[ridx], sem2).wait()

        def issue_scatter():
            pltpu.async_copy(grows, acc.at[ridx], sem2, add=True)

        def make_flush(table_ref):
            def _do_batch():
                pltpu.async_copy(table_ref.at[cidx], grows, sem).wait()

                def scale(j16, _):
                    vv = cvb[pl.ds(j16 * LANES, LANES)]
                    for l in range(LANES):
                        j = j16 * LANES + l
                        v = vv[l]
                        for q in range(C // LANES):
                            d = pl.ds(q * LANES, LANES)
                            grows[j, d] = grows[j, d] * v
                    return 0
                lax.fori_loop(0, BATCH // LANES, scale, 0)
                issue_scatter()

            def flush_full(off):
                drain_scatter()
                for q in range(BATCH // LANES):
                    d = pl.ds(q * LANES, LANES)
                    cidx[d] = ccb[d]
                    ridx[d] = crb[d]
                _do_batch()
                lo16 = pl.ds(0, LANES)
                hi16 = pl.ds(BATCH, LANES)
                crb[lo16] = crb[hi16]
                ccb[lo16] = ccb[hi16]
                cvb[lo16] = cvb[hi16]
                return off - BATCH

            def flush_tail(off):
                drain_scatter()
                for q in range(BATCH // LANES):
                    d = pl.ds(q * LANES, LANES)
                    lane = q * LANES + lax.iota(jnp.int32, LANES)
                    valid = lane < off
                    cidx[d] = jnp.where(valid, ccb[d], 0)
                    ridx[d] = jnp.where(valid, crb[d], rr)
                _do_batch()
                return 0

            return flush_full, flush_tail

        def round_body(rnd, _):
            lo = (2 * rnd + c) * rr
            def zfill(i, _):
                for q in range(C // LANES):
                    grows[i, pl.ds(q * LANES, LANES)] = jnp.zeros(
                        (LANES,), jnp.float32)
                return 0
            lax.fori_loop(0, BATCH, zfill, 0)
            for t in range(nwb):
                base = s * rpt + t * BATCH
                pltpu.sync_copy(grows, acc.at[pl.ds(base, BATCH)])
            plsc.subcore_barrier()

            trash = jnp.full((LANES,), rr, jnp.int32)
            for q in range(BATCH // LANES):
                ridx[pl.ds(q * LANES, LANES)] = trash
            issue_scatter()

            for m in range(num_mats):
                rec_hbm, table_hbm = mat_refs[2 * m:2 * m + 2]
                npt = npts[m]
                flush_full, flush_tail = make_flush(table_hbm)

                def chunk_body(ch, off, rec_hbm=rec_hbm, npt=npt,
                               flush_full=flush_full):
                    base3 = (s * npt + ch * CHUNK) * 3
                    pltpu.sync_copy(rec_hbm.at[pl.ds(base3, CHUNK * 3)],
                                    sbuf)

                    def vec_body(g, off):
                        idxv = iota3 + g * (LANES * 3)
                        r = plsc.load_gather(sbuf, [idxv])
                        cv = plsc.load_gather(sbuf, [idxv + 1])
                        vv = plsc.load_gather(sbuf, [idxv + 2])
                        mask = (r >= lo) & (r < lo + rr)
                        mi = jnp.where(mask, 1, 0).astype(jnp.int32)
                        incl = plsc.cumsum(mi)
                        dest = (off + incl) - mi
                        plsc.store_scatter(crb, [dest], r - lo, mask=mask)
                        plsc.store_scatter(ccb, [dest], cv, mask=mask)
                        plsc.store_scatter(cvb, [dest],
                                           plsc.bitcast(vv, jnp.float32),
                                           mask=mask)
                        off = off + incl[LANES - 1]
                        return lax.cond(off >= BATCH, flush_full,
                                        lambda o: o, off)
                    return lax.fori_loop(0, CHUNK // LANES, vec_body, off)

                off = lax.fori_loop(0, npt // CHUNK, chunk_body, 0)
                flush_tail(off)
            drain_scatter()
            plsc.subcore_barrier()

            pltpu.sync_copy(acc.at[pl.ds(s * rpt, rpt)],
                            out_hbm.at[pl.ds(lo + s * rpt, rpt)])
            return 0

        lax.fori_loop(0, rounds, round_body, 0)

    flat_in = []
    for rows, cols, vals, table in mats:
        rec = jnp.stack(
            [rows, cols, jax.lax.bitcast_convert_type(vals, jnp.int32)],
            axis=1).reshape(-1)
        flat_in += [rec, table]

    fn = pl.kernel(
        body,
        mesh=mesh,
        compiler_params=pltpu.CompilerParams(needs_layout_passes=False),
        out_type=jax.ShapeDtypeStruct((n_pad, C), jnp.float32),
        scratch_types=[
            pltpu.VMEM((CHUNK * 3,), jnp.int32),
            pltpu.VMEM((CBUF,), jnp.int32),
            pltpu.VMEM((CBUF,), jnp.int32),
            pltpu.VMEM((CBUF,), jnp.float32),
            pltpu.VMEM((BATCH,), jnp.int32),
            pltpu.VMEM((BATCH,), jnp.int32),
            pltpu.VMEM((BATCH, C), jnp.float32),
            pltpu.VMEM_SHARED((rr + 8, C), jnp.float32),
            pltpu.SemaphoreType.DMA,
            pltpu.SemaphoreType.DMA,
        ],
    )
    out = fn(*flat_in)
    return out[:n_out]


def _mm_body(nout, relu_in, *refs):
    x_ref = refs[0]
    w_refs = refs[1:1 + nout]
    o_refs = refs[1 + nout:]
    x = x_ref[...]
    if relu_in:
        x = jnp.maximum(x, 0.0)
    for w_ref, o_ref in zip(w_refs, o_refs):
        o_ref[...] = jnp.dot(x, w_ref[...],
                             preferred_element_type=jnp.float32)


def _mm_multi(x, ws, relu_in=False, block_rows=2048):
    n = x.shape[0]
    n_pad = ((n + block_rows - 1) // block_rows) * block_rows
    if n_pad != n:
        x = jnp.pad(x, ((0, n_pad - n), (0, 0)))
    k = len(ws)
    outs = pl.pallas_call(
        functools.partial(_mm_body, k, relu_in),
        grid=(n_pad // block_rows,),
        in_specs=([pl.BlockSpec((block_rows, C), lambda i: (i, 0))]
                  + [pl.BlockSpec((C, C), lambda i: (0, 0))
                     for _ in range(k)]),
        out_specs=[pl.BlockSpec((block_rows, C), lambda i: (i, 0))
                   for _ in range(k)],
        out_shape=[jax.ShapeDtypeStruct((n_pad, C), jnp.float32)
                   for _ in range(k)],
    )(x, *ws)
    if n_pad != n:
        outs = [o[:n] for o in outs]
    return outs


def _head_body(x0_ref, m_ref, w_ref, b_ref, o_ref):
    x0 = jnp.maximum(x0_ref[...], 0.0)
    emb = jnp.dot(x0, w_ref[...],
                  preferred_element_type=jnp.float32) + b_ref[...]
    m = m_ref[...]
    counts = jnp.sum(m, axis=1, keepdims=True)
    pooled = jnp.dot(m, emb, preferred_element_type=jnp.float32)
    o_ref[...] = jnp.where(counts > 0, pooled / jnp.maximum(counts, 1.0), 0.0)


def _head(x0_pre, masks_f32, lin_w, lin_b):
    n = x0_pre.shape[0]
    return pl.pallas_call(
        _head_body,
        in_specs=[
            pl.BlockSpec((n, C), lambda: (0, 0)),
            pl.BlockSpec((NREG, n), lambda: (0, 0)),
            pl.BlockSpec((C, EMB), lambda: (0, 0)),
            pl.BlockSpec((1, EMB), lambda: (0, 0)),
        ],
        out_specs=pl.BlockSpec((NREG, EMB), lambda: (0, 0)),
        out_shape=jax.ShapeDtypeStruct((NREG, EMB), jnp.float32),
    )(x0_pre, masks_f32, lin_w, lin_b.reshape(1, EMB))


RR0, ROUNDS0 = 6144, 1
RR1, ROUNDS1 = 12288, 14


def kernel(x_0, x_1, x_2, incidence_1_indices, incidence_1_values,
           incidence_1_norm_indices, incidence_1_norm_values,
           incidence_2_indices, incidence_2_values,
           incidence_2_norm_indices, incidence_2_norm_values,
           adjacency_up_0_norm_indices, adjacency_up_0_norm_values,
           adjacency_up_1_norm_indices, adjacency_up_1_norm_values,
           adjacency_down_1_norm_indices, adjacency_down_1_norm_values,
           adjacency_down_2_norm_indices, adjacency_down_2_norm_values,
           masks_ip, params):
    p0 = {k[3:]: v for k, v in params.items() if k.startswith('l0_')}
    p1 = {k[3:]: v for k, v in params.items() if k.startswith('l1_')}

    def rcv(idx, vals):
        return jnp.asarray(idx[0]), jnp.asarray(idx[1]), vals

    a00 = rcv(adjacency_up_0_norm_indices, adjacency_up_0_norm_values)
    b1 = rcv(incidence_1_indices, incidence_1_values)
    b1n = rcv(incidence_1_norm_indices, incidence_1_norm_values)
    a1d = rcv(adjacency_down_1_norm_indices, adjacency_down_1_norm_values)
    a1u = rcv(adjacency_up_1_norm_indices, adjacency_up_1_norm_values)
    b2 = rcv(incidence_2_indices, incidence_2_values)

    t00, t01 = _mm_multi(x_0, [p0['W00'], p0['W01']])
    t10, t11d, t11u = _mm_multi(x_1, [p0['W10'], p0['W11d'], p0['W11u']])
    t21, = _mm_multi(x_2, [p0['W21']])

    o0 = _sc_spmm_call([a00 + (t00,), b1 + (t10,)], N0, RR0, ROUNDS0)
    o1 = _sc_spmm_call([b1n + (t01,), a1d + (t11d,), a1u + (t11u,),
                        b2 + (t21,)], N1, RR1, ROUNDS1)

    s00, = _mm_multi(o0, [p1['W00']], relu_in=True)
    s10, = _mm_multi(o1, [p1['W10']], relu_in=True)
    o0b = _sc_spmm_call([a00 + (s00,), b1 + (s10,)], N0, RR0, ROUNDS0)

    return _head(o0b, masks_ip.astype(jnp.float32), params['lin0_w'],
                 params['lin0_b'])


# speedup vs baseline: 6.0875x; 1.8089x over previous
import functools

import jax
import jax.numpy as jnp
from jax import lax
from jax.experimental import pallas as pl
from jax.experimental.pallas import tpu as pltpu
from jax.experimental.pallas import tpu_sc as plsc

N0, N1, N2 = 10000, 320000, 160000
C, EMB, NREG = 128, 32, 64

LANES = 16
BATCH = 128
CBUF = 160
CHUNK = 2000


def _sc_spmm_call(mats, n_out, rr, rounds):
    n_pad = 2 * rounds * rr
    num_mats = len(mats)
    npts = []
    for rows, cols, vals, table in mats:
        nnz = rows.shape[0]
        assert nnz % (16 * CHUNK) == 0, nnz
        npts.append(nnz // 16)

    mesh = plsc.VectorSubcoreMesh(core_axis_name="c", subcore_axis_name="s",
                                  num_cores=2, num_subcores=16)
    rpt = rr // 16
    nwb = rpt // BATCH

    def body(*refs):
        mat_refs = refs[:4 * num_mats]
        out_hbm = refs[4 * num_mats]
        (rbuf, cbuf, vbuf, crb, ccb, cvb, ridx, cidx, grows,
         acc, sem, sem2) = refs[4 * num_mats + 1:]
        c = lax.axis_index("c")
        s = lax.axis_index("s")

        def drain_scatter():
            pltpu.make_async_copy(grows, acc.at[ridx], sem2).wait()

        def issue_scatter():
            pltpu.async_copy(grows, acc.at[ridx], sem2, add=True)

        def make_flush(table_ref):
            def _do_batch():
                pltpu.async_copy(table_ref.at[cidx], grows, sem).wait()

                def scale(j16, _):
                    vv = cvb[pl.ds(j16 * LANES, LANES)]
                    for l in range(LANES):
                        j = j16 * LANES + l
                        v = vv[l]
                        for q in range(C // LANES):
                            d = pl.ds(q * LANES, LANES)
                            grows[j, d] = grows[j, d] * v
                    return 0
                lax.fori_loop(0, BATCH // LANES, scale, 0)
                issue_scatter()

            def flush_full(off):
                drain_scatter()
                for q in range(BATCH // LANES):
                    d = pl.ds(q * LANES, LANES)
                    cidx[d] = ccb[d]
                    ridx[d] = crb[d]
                _do_batch()
                lo16 = pl.ds(0, LANES)
                hi16 = pl.ds(BATCH, LANES)
                crb[lo16] = crb[hi16]
                ccb[lo16] = ccb[hi16]
                cvb[lo16] = cvb[hi16]
                return off - BATCH

            def flush_tail(off):
                drain_scatter()
                for q in range(BATCH // LANES):
                    d = pl.ds(q * LANES, LANES)
                    lane = q * LANES + lax.iota(jnp.int32, LANES)
                    valid = lane < off
                    cidx[d] = jnp.where(valid, ccb[d], 0)
                    ridx[d] = jnp.where(valid, crb[d], rr)
                _do_batch()
                return 0

            return flush_full, flush_tail

        def round_body(rnd, _):
            lo = (2 * rnd + c) * rr
            def zfill(i, _):
                for q in range(C // LANES):
                    grows[i, pl.ds(q * LANES, LANES)] = jnp.zeros(
                        (LANES,), jnp.float32)
                return 0
            lax.fori_loop(0, BATCH, zfill, 0)
            for t in range(nwb):
                base = s * rpt + t * BATCH
                pltpu.sync_copy(grows, acc.at[pl.ds(base, BATCH)])
            plsc.subcore_barrier()

            trash = jnp.full((LANES,), rr, jnp.int32)
            for q in range(BATCH // LANES):
                ridx[pl.ds(q * LANES, LANES)] = trash
            issue_scatter()

            for m in range(num_mats):
                rows_hbm, cols_hbm, vals_hbm, table_hbm = \
                    mat_refs[4 * m:4 * m + 4]
                npt = npts[m]
                flush_full, flush_tail = make_flush(table_hbm)

                def chunk_body(ch, off, rows_hbm=rows_hbm,
                               cols_hbm=cols_hbm, vals_hbm=vals_hbm,
                               npt=npt, flush_full=flush_full):
                    base = s * npt + ch * CHUNK
                    d = pl.ds(base, CHUNK)
                    pltpu.async_copy(rows_hbm.at[d], rbuf, sem)
                    pltpu.async_copy(cols_hbm.at[d], cbuf, sem)
                    pltpu.async_copy(vals_hbm.at[d], vbuf, sem)
                    pltpu.make_async_copy(rows_hbm.at[d], rbuf, sem).wait()
                    pltpu.make_async_copy(cols_hbm.at[d], cbuf, sem).wait()
                    pltpu.make_async_copy(vals_hbm.at[d], vbuf, sem).wait()

                    def vec_body(g, off):
                        dg = pl.ds(g * LANES, LANES)
                        r = rbuf[dg]
                        mask = (r >= lo) & (r < lo + rr)
                        mi = jnp.where(mask, 1, 0).astype(jnp.int32)
                        incl = plsc.cumsum(mi)
                        dest = (off + incl) - mi
                        plsc.store_scatter(crb, [dest], r - lo, mask=mask)
                        plsc.store_scatter(ccb, [dest], cbuf[dg], mask=mask)
                        plsc.store_scatter(cvb, [dest], vbuf[dg], mask=mask)
                        off = off + incl[LANES - 1]
                        return lax.cond(off >= BATCH, flush_full,
                                        lambda o: o, off)
                    return lax.fori_loop(0, CHUNK // LANES, vec_body, off)

                off = lax.fori_loop(0, npt // CHUNK, chunk_body, 0)
                flush_tail(off)
            drain_scatter()
            plsc.subcore_barrier()

            pltpu.sync_copy(acc.at[pl.ds(s * rpt, rpt)],
                            out_hbm.at[pl.ds(lo + s * rpt, rpt)])
            return 0

        lax.fori_loop(0, rounds, round_body, 0)

    flat_in = []
    for rows, cols, vals, table in mats:
        flat_in += [rows, cols, vals, table]

    fn = pl.kernel(
        body,
        mesh=mesh,
        compiler_params=pltpu.CompilerParams(needs_layout_passes=False),
        out_type=jax.ShapeDtypeStruct((n_pad, C), jnp.float32),
        scratch_types=[
            pltpu.VMEM((CHUNK,), jnp.int32),
            pltpu.VMEM((CHUNK,), jnp.int32),
            pltpu.VMEM((CHUNK,), jnp.float32),
            pltpu.VMEM((CBUF,), jnp.int32),
            pltpu.VMEM((CBUF,), jnp.int32),
            pltpu.VMEM((CBUF,), jnp.float32),
            pltpu.VMEM((BATCH,), jnp.int32),
            pltpu.VMEM((BATCH,), jnp.int32),
            pltpu.VMEM((BATCH, C), jnp.float32),
            pltpu.VMEM_SHARED((rr + 8, C), jnp.float32),
            pltpu.SemaphoreType.DMA,
            pltpu.SemaphoreType.DMA,
        ],
    )
    out = fn(*flat_in)
    return out[:n_out]


def _mm_body(nout, relu_in, *refs):
    x_ref = refs[0]
    w_refs = refs[1:1 + nout]
    o_refs = refs[1 + nout:]
    x = x_ref[...]
    if relu_in:
        x = jnp.maximum(x, 0.0)
    for w_ref, o_ref in zip(w_refs, o_refs):
        o_ref[...] = jnp.dot(x, w_ref[...],
                             preferred_element_type=jnp.float32)


def _mm_multi(x, ws, relu_in=False, block_rows=2048):
    n = x.shape[0]
    n_pad = ((n + block_rows - 1) // block_rows) * block_rows
    if n_pad != n:
        x = jnp.pad(x, ((0, n_pad - n), (0, 0)))
    k = len(ws)
    outs = pl.pallas_call(
        functools.partial(_mm_body, k, relu_in),
        grid=(n_pad // block_rows,),
        in_specs=([pl.BlockSpec((block_rows, C), lambda i: (i, 0))]
                  + [pl.BlockSpec((C, C), lambda i: (0, 0))
                     for _ in range(k)]),
        out_specs=[pl.BlockSpec((block_rows, C), lambda i: (i, 0))
                   for _ in range(k)],
        out_shape=[jax.ShapeDtypeStruct((n_pad, C), jnp.float32)
                   for _ in range(k)],
    )(x, *ws)
    if n_pad != n:
        outs = [o[:n] for o in outs]
    return outs


def _head_body(x0_ref, m_ref, w_ref, b_ref, o_ref):
    x0 = jnp.maximum(x0_ref[...], 0.0)
    emb = jnp.dot(x0, w_ref[...],
                  preferred_element_type=jnp.float32) + b_ref[...]
    m = m_ref[...]
    counts = jnp.sum(m, axis=1, keepdims=True)
    pooled = jnp.dot(m, emb, preferred_element_type=jnp.float32)
    o_ref[...] = jnp.where(counts > 0, pooled / jnp.maximum(counts, 1.0), 0.0)


def _head(x0_pre, masks_f32, lin_w, lin_b):
    n = x0_pre.shape[0]
    return pl.pallas_call(
        _head_body,
        in_specs=[
            pl.BlockSpec((n, C), lambda: (0, 0)),
            pl.BlockSpec((NREG, n), lambda: (0, 0)),
            pl.BlockSpec((C, EMB), lambda: (0, 0)),
            pl.BlockSpec((1, EMB), lambda: (0, 0)),
        ],
        out_specs=pl.BlockSpec((NREG, EMB), lambda: (0, 0)),
        out_shape=jax.ShapeDtypeStruct((NREG, EMB), jnp.float32),
    )(x0_pre, masks_f32, lin_w, lin_b.reshape(1, EMB))


RR0, ROUNDS0 = 6144, 1
RR1, ROUNDS1 = 12288, 14


def kernel(x_0, x_1, x_2, incidence_1_indices, incidence_1_values,
           incidence_1_norm_indices, incidence_1_norm_values,
           incidence_2_indices, incidence_2_values,
           incidence_2_norm_indices, incidence_2_norm_values,
           adjacency_up_0_norm_indices, adjacency_up_0_norm_values,
           adjacency_up_1_norm_indices, adjacency_up_1_norm_values,
           adjacency_down_1_norm_indices, adjacency_down_1_norm_values,
           adjacency_down_2_norm_indices, adjacency_down_2_norm_values,
           masks_ip, params):
    p0 = {k[3:]: v for k, v in params.items() if k.startswith('l0_')}
    p1 = {k[3:]: v for k, v in params.items() if k.startswith('l1_')}

    def rcv(idx, vals):
        return jnp.asarray(idx[0]), jnp.asarray(idx[1]), vals

    a00 = rcv(adjacency_up_0_norm_indices, adjacency_up_0_norm_values)
    b1 = rcv(incidence_1_indices, incidence_1_values)
    b1n = rcv(incidence_1_norm_indices, incidence_1_norm_values)
    a1d = rcv(adjacency_down_1_norm_indices, adjacency_down_1_norm_values)
    a1u = rcv(adjacency_up_1_norm_indices, adjacency_up_1_norm_values)
    b2 = rcv(incidence_2_indices, incidence_2_values)

    t00, t01 = _mm_multi(x_0, [p0['W00'], p0['W01']])
    t10, t11d, t11u = _mm_multi(x_1, [p0['W10'], p0['W11d'], p0['W11u']])
    t21, = _mm_multi(x_2, [p0['W21']])

    o0 = _sc_spmm_call([a00 + (t00,), b1 + (t10,)], N0, RR0, ROUNDS0)
    o1 = _sc_spmm_call([b1n + (t01,), a1d + (t11d,), a1u + (t11u,),
                        b2 + (t21,)], N1, RR1, ROUNDS1)

    s00, = _mm_multi(o0, [p1['W00']], relu_in=True)
    s10, = _mm_multi(o1, [p1['W10']], relu_in=True)
    o0b = _sc_spmm_call([a00 + (s00,), b1 + (s10,)], N0, RR0, ROUNDS0)

    return _head(o0b, masks_ip.astype(jnp.float32), params['lin0_w'],
                 params['lin0_b'])
